# baseline (speedup 1.0000x reference)
import functools

import jax
import jax.numpy as jnp
from jax import lax
from jax.experimental import pallas as pl
from jax.experimental.pallas import tpu as pltpu
from jax.experimental.pallas import tpu_sc as plsc

N = 10000
D = 128
E = 320000
G = 200

C = 128
SC_TILES = 16
NBUF = 2
ISLOT = 4
NCH = 160
NGRP = NCH // 4
PER_TILE = NCH * C
EP = PER_TILE * SC_TILES
NP = 10112
ZROWS = NP // SC_TILES


def _sc_aggregate(x1, src1, dst1, x2, src2, dst2, zeros_np):
    mesh = plsc.VectorSubcoreMesh(core_axis_name="c", subcore_axis_name="s")

    @functools.partial(
        pl.kernel,
        mesh=mesh,
        out_type=jax.ShapeDtypeStruct((2, NP, D), jnp.float32),
        scratch_types=[
            pltpu.VMEM_SHARED((NP, D), jnp.float32),
            pltpu.VMEM((ISLOT, C), jnp.int32),
            pltpu.VMEM((ISLOT, C), jnp.int32),
            pltpu.VMEM((NBUF * C, D), jnp.float32),
            pltpu.SemaphoreType.DMA,
            pltpu.SemaphoreType.DMA,
            pltpu.SemaphoreType.DMA,
            pltpu.SemaphoreType.DMA,
            pltpu.SemaphoreType.DMA,
            pltpu.SemaphoreType.DMA,
            pltpu.SemaphoreType.DMA,
            pltpu.SemaphoreType.DMA,
        ],
    )
    def k(x1_h, s1_h, d1_h, x2_h, s2_h, d2_h, z_h, out_h,
          agg_sh, sidx, didx, rows,
          g0, g1, s0, s1, i0, i1, i2, i3):
        c = lax.axis_index("c")
        s = lax.axis_index("s")
        gsems = (g0, g1)
        ssems = (s0, s1)
        isems = (i0, i1, i2, i3)

        pltpu.sync_copy(z_h.at[pl.ds(s * ZROWS, ZROWS)],
                        agg_sh.at[pl.ds(s * ZROWS, ZROWS)])
        plsc.subcore_barrier()

        def run(x_h, src_h, dst_h):
            def fire_idx(j, slot):
                pltpu.async_copy(src_h.at[s, j], sidx.at[slot], isems[slot])
                pltpu.async_copy(dst_h.at[s, j], didx.at[slot], isems[slot])

            def drain_idx(slot):
                for _ in range(2):
                    pltpu.make_async_copy(src_h.at[s, 0], sidx.at[slot],
                                          isems[slot]).wait()

            def fire_gather(slot, b):
                pltpu.async_copy(x_h.at[sidx.at[slot]],
                                 rows.at[pl.ds(b * C, C)], gsems[b])

            def drain_rows(b, sem):
                pltpu.make_async_copy(x_h.at[pl.ds(0, C)],
                                      rows.at[pl.ds(b * C, C)], sem).wait()

            for j in range(ISLOT):
                fire_idx(j, j)
            for b in range(NBUF):
                drain_idx(b)
                fire_gather(b, b)

            def body(jg, carry):
                for k in range(4):
                    j = jg * 4 + k
                    b = k % 2
                    drain_rows(b, gsems[b])
                    pltpu.async_copy(rows.at[pl.ds(b * C, C)],
                                     agg_sh.at[didx.at[k]], ssems[b],
                                     add=True)
                    drain_rows(b, ssems[b])

                    @pl.when(j + ISLOT < NCH)
                    def _():
                        fire_idx(j + ISLOT, k)

                    @pl.when(j + NBUF < NCH)
                    def _():
                        drain_idx((k + NBUF) % ISLOT)
                        fire_gather((k + NBUF) % ISLOT, b)
                return carry
            lax.fori_loop(0, NGRP, body, 0)

        @pl.when(c == 0)
        def _():
            run(x1_h, s1_h, d1_h)

        @pl.when(c == 1)
        def _():
            run(x2_h, s2_h, d2_h)

        plsc.subcore_barrier()
        pltpu.sync_copy(agg_sh.at[pl.ds(s * ZROWS, ZROWS)],
                        out_h.at[c, pl.ds(s * ZROWS, ZROWS)])

    return k(x1, src1, dst1, x2, src2, dst2, zeros_np)


BN = 1000
NB = N // BN


def _tc_head(agg, x, Wg, bg, Wrg, brg, W1g, b1g):
    def body(agg_ref, x_ref, w_ref, b_ref, wr_ref, br_ref, w1_ref, b1_ref,
             s_ref):
        j = pl.program_id(1)
        a = agg_ref[0]
        xb = x_ref[0]
        h = jnp.maximum(
            jnp.dot(a, w_ref[0], preferred_element_type=jnp.float32)
            + b_ref[0], 0.0)
        r = jnp.maximum(
            jnp.dot(xb, wr_ref[0], preferred_element_type=jnp.float32)
            + br_ref[0], 0.0)
        t = jnp.maximum(
            jnp.dot(h + r, w1_ref[0], preferred_element_type=jnp.float32)
            + b1_ref[0], 0.0)

        @pl.when(j == 0)
        def _():
            s_ref[...] = jnp.zeros_like(s_ref)

        s_ref[0] += jnp.sum(t, axis=0, keepdims=True)

    return pl.pallas_call(
        body,
        grid=(2, NB),
        in_specs=[
            pl.BlockSpec((1, BN, D), lambda g, j: (g, j, 0)),
            pl.BlockSpec((1, BN, D), lambda g, j: (g, j, 0)),
            pl.BlockSpec((1, D, D), lambda g, j: (g, 0, 0)),
            pl.BlockSpec((1, 1, D), lambda g, j: (g, 0, 0)),
            pl.BlockSpec((1, D, D), lambda g, j: (g, 0, 0)),
            pl.BlockSpec((1, 1, D), lambda g, j: (g, 0, 0)),
            pl.BlockSpec((1, D, D), lambda g, j: (g, 0, 0)),
            pl.BlockSpec((1, 1, D), lambda g, j: (g, 0, 0)),
        ],
        out_specs=pl.BlockSpec((1, 1, D), lambda g, j: (g, 0, 0)),
        out_shape=jax.ShapeDtypeStruct((2, 1, D), jnp.float32),
    )(agg, x, Wg, bg, Wrg, brg, W1g, b1g)


def kernel(node_feats_1, edge_index_1, edge_feats_1,
           node_feats_2, edge_index_2, edge_feats_2,
           W_g1, b_g1, Wr_g1, br_g1, W1_r1, b1_r1, W2_r1, b2_r1,
           W_g2, b_g2, Wr_g2, br_g2, W1_r2, b1_r2, W2_r2, b2_r2,
           Wp, bp):
    pad = EP - E

    def prep(ei):
        src = jnp.concatenate(
            [ei[0], jnp.zeros((pad,), jnp.int32)]).reshape(SC_TILES, NCH, C)
        dst = jnp.concatenate(
            [ei[1], jnp.full((pad,), N, jnp.int32)]).reshape(SC_TILES, NCH, C)
        return src, dst

    src1, dst1 = prep(edge_index_1)
    src2, dst2 = prep(edge_index_2)
    zeros_np = jnp.zeros((NP, D), jnp.float32)

    agg = _sc_aggregate(node_feats_1, src1, dst1,
                        node_feats_2, src2, dst2, zeros_np)

    x = jnp.stack([node_feats_1, node_feats_2])
    Wg = jnp.stack([W_g1, W_g2])
    bg = jnp.stack([b_g1, b_g2]).reshape(2, 1, D)
    Wrg = jnp.stack([Wr_g1, Wr_g2]).reshape(2, D, D)
    brg = jnp.stack([br_g1, br_g2]).reshape(2, 1, D)
    W1g = jnp.stack([W1_r1, W1_r2])
    b1g = jnp.stack([b1_r1, b1_r2]).reshape(2, 1, D)

    s = _tc_head(agg, x, Wg, bg, Wrg, brg, W1g, b1g)

    g_vec = (s[0, 0] @ W2_r1 + N * b2_r1) + (s[1, 0] @ W2_r2 + N * b2_r2)
    out = g_vec @ Wp + bp
    return out.reshape(-1)


# speedup vs baseline: 1.5569x; 1.5569x over previous
import functools

import jax
import jax.numpy as jnp
from jax import lax
from jax.experimental import pallas as pl
from jax.experimental.pallas import tpu as pltpu
from jax.experimental.pallas import tpu_sc as plsc

N = 10000
D = 128
E = 320000
G = 200

C = 56
SC_TILES = 16
RING = 6
GLA = 4
NCH = 360
NGRP = NCH // RING
PER_TILE = NCH * C
EP = PER_TILE * SC_TILES
NP = 10112
ZROWS = NP // SC_TILES


def _sc_aggregate(x1, e1, x2, e2, zeros_np):
    mesh = plsc.VectorSubcoreMesh(core_axis_name="c", subcore_axis_name="s")

    @functools.partial(
        pl.kernel,
        mesh=mesh,
        out_type=jax.ShapeDtypeStruct((2, NP, D), jnp.float32),
        scratch_types=[
            pltpu.VMEM_SHARED((NP, D), jnp.float32),
            pltpu.VMEM((RING, 2, C), jnp.int32),
            pltpu.VMEM((RING * C, D), jnp.float32),
            pltpu.SemaphoreType.DMA,
            pltpu.SemaphoreType.DMA,
            pltpu.SemaphoreType.DMA,
            pltpu.SemaphoreType.DMA,
            pltpu.SemaphoreType.DMA,
            pltpu.SemaphoreType.DMA,
            pltpu.SemaphoreType.DMA,
            pltpu.SemaphoreType.DMA,
            pltpu.SemaphoreType.DMA,
            pltpu.SemaphoreType.DMA,
            pltpu.SemaphoreType.DMA,
            pltpu.SemaphoreType.DMA,
            pltpu.SemaphoreType.DMA,
        ],
    )
    def k(x1_h, e1_h, x2_h, e2_h, z_h, out_h,
          agg_sh, idx, rows,
          g0, g1, g2, g3, g4, g5, i0, i1, i2, i3, i4, i5, ssem):
        c = lax.axis_index("c")
        s = lax.axis_index("s")
        gsems = (g0, g1, g2, g3, g4, g5)
        isems = (i0, i1, i2, i3, i4, i5)

        pltpu.sync_copy(z_h.at[pl.ds(s * ZROWS, ZROWS)],
                        agg_sh.at[pl.ds(s * ZROWS, ZROWS)])
        plsc.subcore_barrier()

        def run(x_h, e_h):
            def fire_idx(j, sl):
                pltpu.async_copy(e_h.at[s, j], idx.at[sl], isems[sl])

            def drain_idx(sl):
                pltpu.make_async_copy(e_h.at[s, 0], idx.at[sl],
                                      isems[sl]).wait()

            def fire_gather(sl):
                pltpu.async_copy(x_h.at[idx.at[sl, 0]],
                                 rows.at[pl.ds(sl * C, C)], gsems[sl])

            def drain_rows(sl, sem):
                pltpu.make_async_copy(x_h.at[pl.ds(0, C)],
                                      rows.at[pl.ds(sl * C, C)], sem).wait()

            for j in range(RING):
                fire_idx(j, j)
            for sl in range(GLA):
                drain_idx(sl)
                fire_gather(sl)

            def body(jg, carry):
                for k in range(RING):
                    j = jg * RING + k
                    drain_rows(k, gsems[k])
                    pltpu.async_copy(rows.at[pl.ds(k * C, C)],
                                     agg_sh.at[idx.at[k, 1]], ssem,
                                     add=True)
                    drain_rows(k, ssem)

                    @pl.when(j + RING < NCH)
                    def _():
                        fire_idx(j + RING, k)

                    @pl.when(j + GLA < NCH)
                    def _():
                        sl2 = (k + GLA) % RING
                        drain_idx(sl2)
                        fire_gather(sl2)
                return carry
            lax.fori_loop(0, NGRP, body, 0)

        @pl.when(c == 0)
        def _():
            run(x1_h, e1_h)

        @pl.when(c == 1)
        def _():
            run(x2_h, e2_h)

        plsc.subcore_barrier()
        pltpu.sync_copy(agg_sh.at[pl.ds(s * ZROWS, ZROWS)],
                        out_h.at[c, pl.ds(s * ZROWS, ZROWS)])

    return k(x1, e1, x2, e2, zeros_np)


BN = 1000
NB = N // BN


def _tc_head(agg, x, Wg, bg, Wrg, brg, W1g, b1g):
    def body(agg_ref, x_ref, w_ref, b_ref, wr_ref, br_ref, w1_ref, b1_ref,
             s_ref):
        j = pl.program_id(1)
        a = agg_ref[0]
        xb = x_ref[0]
        h = jnp.maximum(
            jnp.dot(a, w_ref[0], preferred_element_type=jnp.float32)
            + b_ref[0], 0.0)
        r = jnp.maximum(
            jnp.dot(xb, wr_ref[0], preferred_element_type=jnp.float32)
            + br_ref[0], 0.0)
        t = jnp.maximum(
            jnp.dot(h + r, w1_ref[0], preferred_element_type=jnp.float32)
            + b1_ref[0], 0.0)

        @pl.when(j == 0)
        def _():
            s_ref[...] = jnp.zeros_like(s_ref)

        s_ref[0] += jnp.sum(t, axis=0, keepdims=True)

    return pl.pallas_call(
        body,
        grid=(2, NB),
        in_specs=[
            pl.BlockSpec((1, BN, D), lambda g, j: (g, j, 0)),
            pl.BlockSpec((1, BN, D), lambda g, j: (g, j, 0)),
            pl.BlockSpec((1, D, D), lambda g, j: (g, 0, 0)),
            pl.BlockSpec((1, 1, D), lambda g, j: (g, 0, 0)),
            pl.BlockSpec((1, D, D), lambda g, j: (g, 0, 0)),
            pl.BlockSpec((1, 1, D), lambda g, j: (g, 0, 0)),
            pl.BlockSpec((1, D, D), lambda g, j: (g, 0, 0)),
            pl.BlockSpec((1, 1, D), lambda g, j: (g, 0, 0)),
        ],
        out_specs=pl.BlockSpec((1, 1, D), lambda g, j: (g, 0, 0)),
        out_shape=jax.ShapeDtypeStruct((2, 1, D), jnp.float32),
    )(agg, x, Wg, bg, Wrg, brg, W1g, b1g)


def kernel(node_feats_1, edge_index_1, edge_feats_1,
           node_feats_2, edge_index_2, edge_feats_2,
           W_g1, b_g1, Wr_g1, br_g1, W1_r1, b1_r1, W2_r1, b2_r1,
           W_g2, b_g2, Wr_g2, br_g2, W1_r2, b1_r2, W2_r2, b2_r2,
           Wp, bp):
    pad = EP - E

    def prep(ei):
        src = jnp.concatenate(
            [ei[0], jnp.zeros((pad,), jnp.int32)]).reshape(SC_TILES, NCH, C)
        dst = jnp.concatenate(
            [ei[1], jnp.full((pad,), N, jnp.int32)]).reshape(SC_TILES, NCH, C)
        return jnp.stack([src, dst], axis=2)

    e1 = prep(edge_index_1)
    e2 = prep(edge_index_2)
    zeros_np = jnp.zeros((NP, D), jnp.float32)

    agg = _sc_aggregate(node_feats_1, e1, node_feats_2, e2, zeros_np)

    x = jnp.stack([node_feats_1, node_feats_2])
    Wg = jnp.stack([W_g1, W_g2])
    bg = jnp.stack([b_g1, b_g2]).reshape(2, 1, D)
    Wrg = jnp.stack([Wr_g1, Wr_g2]).reshape(2, D, D)
    brg = jnp.stack([br_g1, br_g2]).reshape(2, 1, D)
    W1g = jnp.stack([W1_r1, W1_r2])
    b1g = jnp.stack([b1_r1, b1_r2]).reshape(2, 1, D)

    s = _tc_head(agg, x, Wg, bg, Wrg, brg, W1g, b1g)

    g_vec = (s[0, 0] @ W2_r1 + N * b2_r1) + (s[1, 0] @ W2_r2 + N * b2_r2)
    out = g_vec @ Wp + bp
    return out.reshape(-1)


# speedup vs baseline: 1.6490x; 1.0592x over previous
import functools

import jax
import jax.numpy as jnp
from jax import lax
from jax.experimental import pallas as pl
from jax.experimental.pallas import tpu as pltpu
from jax.experimental.pallas import tpu_sc as plsc

N = 10000
D = 128
E = 320000
G = 200

C = 56
SC_TILES = 16
RING = 6
GLA = 4
NCH = 360
NGRP = NCH // RING
PER_TILE = NCH * C
EP = PER_TILE * SC_TILES
NP = 10112
ZROWS = NP // SC_TILES


def _sc_aggregate(x1, e1, x2, e2, zeros_np):
    mesh = plsc.VectorSubcoreMesh(core_axis_name="c", subcore_axis_name="s")

    @functools.partial(
        pl.kernel,
        mesh=mesh,
        out_type=jax.ShapeDtypeStruct((2, NP, D), jnp.float32),
        scratch_types=[
            pltpu.VMEM_SHARED((NP, D), jnp.float32),
            pltpu.VMEM((RING, 2, C), jnp.int32),
            pltpu.VMEM((RING * C, D), jnp.float32),
            pltpu.SemaphoreType.DMA,
            pltpu.SemaphoreType.DMA,
            pltpu.SemaphoreType.DMA,
            pltpu.SemaphoreType.DMA,
            pltpu.SemaphoreType.DMA,
            pltpu.SemaphoreType.DMA,
            pltpu.SemaphoreType.DMA,
            pltpu.SemaphoreType.DMA,
            pltpu.SemaphoreType.DMA,
            pltpu.SemaphoreType.DMA,
            pltpu.SemaphoreType.DMA,
            pltpu.SemaphoreType.DMA,
            pltpu.SemaphoreType.DMA,
        ],
    )
    def k(x1_h, e1_h, x2_h, e2_h, z_h, out_h,
          agg_sh, idx, rows,
          g0, g1, g2, g3, g4, g5, i0, i1, i2, i3, i4, i5, ssem):
        c = lax.axis_index("c")
        s = lax.axis_index("s")
        gsems = (g0, g1, g2, g3, g4, g5)
        isems = (i0, i1, i2, i3, i4, i5)

        pltpu.sync_copy(z_h.at[pl.ds(s * ZROWS, ZROWS)],
                        agg_sh.at[pl.ds(s * ZROWS, ZROWS)])
        plsc.subcore_barrier()

        def run(x_h, e_h):
            def fire_idx(j, sl):
                pltpu.async_copy(e_h.at[s, j], idx.at[sl], isems[sl])

            def drain_idx(sl):
                pltpu.make_async_copy(e_h.at[s, 0], idx.at[sl],
                                      isems[sl]).wait()

            def fire_gather(sl):
                pltpu.async_copy(x_h.at[idx.at[sl, 0]],
                                 rows.at[pl.ds(sl * C, C)], gsems[sl])

            def drain_rows(sl, sem):
                pltpu.make_async_copy(x_h.at[pl.ds(0, C)],
                                      rows.at[pl.ds(sl * C, C)], sem).wait()

            for j in range(RING):
                fire_idx(j, j)
            for sl in range(GLA):
                drain_idx(sl)
                fire_gather(sl)

            def body(jg, carry):
                for k in range(RING):
                    j = jg * RING + k
                    drain_rows(k, gsems[k])
                    pltpu.async_copy(rows.at[pl.ds(k * C, C)],
                                     agg_sh.at[idx.at[k, 1]], ssem,
                                     add=True)
                    drain_rows(k, ssem)

                    @pl.when(j + RING < NCH)
                    def _():
                        fire_idx(j + RING, k)

                    @pl.when(j + GLA < NCH)
                    def _():
                        sl2 = (k + GLA) % RING
                        drain_idx(sl2)
                        fire_gather(sl2)
                return carry
            lax.fori_loop(0, NGRP, body, 0)

        @pl.when(c == 0)
        def _():
            run(x1_h, e1_h)

        @pl.when(c == 1)
        def _():
            run(x2_h, e2_h)

        plsc.subcore_barrier()
        pltpu.sync_copy(agg_sh.at[pl.ds(s * ZROWS, ZROWS)],
                        out_h.at[c, pl.ds(s * ZROWS, ZROWS)])

    return k(x1, e1, x2, e2, zeros_np)


BN = 2000
NB = N // BN


def _tc_head(agg, x1, x2, Wg1, bg1, Wr1, br1, W11, b11,
             Wg2, bg2, Wr2, br2, W12, b12,
             W2a, b2a, W2b, b2b, Wp, bp):
    def body(agg_ref, x1_ref, x2_ref,
             wg1, bg1r, wr1, br1r, w11, b11r,
             wg2, bg2r, wr2, br2r, w12, b12r,
             w2a, b2ar, w2b, b2br, wp, bpr,
             out_ref, s1_acc, s2_acc):
        j = pl.program_id(0)

        @pl.when(j == 0)
        def _():
            s1_acc[...] = jnp.zeros_like(s1_acc)
            s2_acc[...] = jnp.zeros_like(s2_acc)

        def graph_block(a, xb, w, b, wr, br, w1, b1):
            h = jnp.maximum(
                jnp.dot(a, w, preferred_element_type=jnp.float32) + b, 0.0)
            r = jnp.maximum(
                jnp.dot(xb, wr, preferred_element_type=jnp.float32) + br, 0.0)
            t = jnp.maximum(
                jnp.dot(h + r, w1, preferred_element_type=jnp.float32) + b1,
                0.0)
            return jnp.sum(t, axis=0, keepdims=True)

        s1_acc[...] += graph_block(agg_ref[0], x1_ref[...], wg1[...],
                                   bg1r[...], wr1[...], br1r[...],
                                   w11[...], b11r[...])
        s2_acc[...] += graph_block(agg_ref[1], x2_ref[...], wg2[...],
                                   bg2r[...], wr2[...], br2r[...],
                                   w12[...], b12r[...])

        @pl.when(j == NB - 1)
        def _():
            g_vec = (jnp.dot(s1_acc[...], w2a[...],
                             preferred_element_type=jnp.float32)
                     + jnp.dot(s2_acc[...], w2b[...],
                               preferred_element_type=jnp.float32)
                     + N * (b2ar[...] + b2br[...]))
            out_ref[...] = (jnp.dot(g_vec, wp[...],
                                    preferred_element_type=jnp.float32)
                            + bpr[...])

    full = lambda *shape: pl.BlockSpec(shape, lambda j: (0,) * len(shape))
    return pl.pallas_call(
        body,
        grid=(NB,),
        in_specs=[
            pl.BlockSpec((2, BN, D), lambda j: (0, j, 0)),
            pl.BlockSpec((BN, D), lambda j: (j, 0)),
            pl.BlockSpec((BN, D), lambda j: (j, 0)),
            full(D, D), full(1, D), full(D, D), full(1, D),
            full(D, D), full(1, D),
            full(D, D), full(1, D), full(D, D), full(1, D),
            full(D, D), full(1, D),
            full(D, G), full(1, G), full(D, G), full(1, G),
            full(G, 1), full(1, 1),
        ],
        out_specs=pl.BlockSpec((1, 1), lambda j: (0, 0)),
        out_shape=jax.ShapeDtypeStruct((1, 1), jnp.float32),
        scratch_shapes=[pltpu.VMEM((1, D), jnp.float32),
                        pltpu.VMEM((1, D), jnp.float32)],
    )(agg, x1, x2, Wg1, bg1, Wr1, br1, W11, b11,
      Wg2, bg2, Wr2, br2, W12, b12, W2a, b2a, W2b, b2b, Wp, bp)


def kernel(node_feats_1, edge_index_1, edge_feats_1,
           node_feats_2, edge_index_2, edge_feats_2,
           W_g1, b_g1, Wr_g1, br_g1, W1_r1, b1_r1, W2_r1, b2_r1,
           W_g2, b_g2, Wr_g2, br_g2, W1_r2, b1_r2, W2_r2, b2_r2,
           Wp, bp):
    pad = EP - E

    def prep(ei):
        src = jnp.concatenate(
            [ei[0], jnp.zeros((pad,), jnp.int32)]).reshape(SC_TILES, NCH, C)
        dst = jnp.concatenate(
            [ei[1], jnp.full((pad,), N, jnp.int32)]).reshape(SC_TILES, NCH, C)
        return jnp.stack([src, dst], axis=2)

    e1 = prep(edge_index_1)
    e2 = prep(edge_index_2)
    zeros_np = jnp.zeros((NP, D), jnp.float32)

    agg = _sc_aggregate(node_feats_1, e1, node_feats_2, e2, zeros_np)

    out = _tc_head(agg, node_feats_1, node_feats_2,
                   W_g1, b_g1.reshape(1, D), Wr_g1, br_g1.reshape(1, D),
                   W1_r1, b1_r1.reshape(1, D),
                   W_g2, b_g2.reshape(1, D), Wr_g2, br_g2.reshape(1, D),
                   W1_r2, b1_r2.reshape(1, D),
                   W2_r1, b2_r1.reshape(1, G), W2_r2, b2_r2.reshape(1, G),
                   Wp, bp.reshape(1, 1))
    return out.reshape(-1)
